# Initial kernel scaffold; baseline (speedup 1.0000x reference)
import jax
import jax.numpy as jnp
from jax.experimental import pallas as pl


def kernel(logits):
    raise NotImplementedError("write your pallas kernel here")


# speedup vs baseline: 1.0496x; 1.0496x over previous
import functools

import jax
import jax.numpy as jnp
import numpy as np
from jax.experimental import pallas as pl
from jax.experimental.pallas import tpu as pltpu

_R, _C = 64, 100000
_BLK = 2048
_CHUNK = 128
_NBLK = (_C + _BLK - 1) // _BLK
_NCHUNK = _BLK // _CHUNK

_KS = (np.uint32(0), np.uint32(42), np.uint32(0x1BD11BDA ^ 0 ^ 42))
_ROT_A = (13, 15, 26, 6)
_ROT_B = (17, 29, 16, 24)
_TINY = np.float32(np.finfo(np.float32).tiny)
_NEG_INF = np.float32(-np.inf)
_ONE_BITS = np.uint32(0x3F800000)
_INT_MAX = np.int32(np.iinfo(np.int32).max)


def _rotl(x, d):
    return jax.lax.shift_left(x, np.uint32(d)) | jax.lax.shift_right_logical(
        x, np.uint32(32 - d)
    )


def _threefry_bits(idx):
    x0 = jnp.zeros_like(idx)
    x1 = idx + _KS[1]
    rots = (_ROT_A, _ROT_B)
    for g in range(5):
        for r in rots[g % 2]:
            x0 = x0 + x1
            x1 = _rotl(x1, r)
            x1 = x1 ^ x0
        x0 = x0 + _KS[(g + 1) % 3]
        x1 = x1 + np.uint32(int(_KS[(g + 2) % 3]) + g + 1 & 0xFFFFFFFF)
    return x0 ^ x1


def _kernel(x_ref, o_ref, vmax_ref, vidx_ref):
    j = pl.program_id(0)

    @pl.when(j == 0)
    def _init():
        vmax_ref[...] = jnp.full((_R, _CHUNK), _NEG_INF, jnp.float32)
        vidx_ref[...] = jnp.zeros((_R, _CHUNK), jnp.int32)

    row_iota = jax.lax.broadcasted_iota(jnp.uint32, (_R, _CHUNK), 0)
    lane_iota = jax.lax.broadcasted_iota(jnp.uint32, (_R, _CHUNK), 1)
    rb = row_iota * np.uint32(_C) + lane_iota
    lane_i32 = jax.lax.broadcasted_iota(jnp.int32, (_R, _CHUNK), 1)

    vm = vmax_ref[...]
    vi = vidx_ref[...]
    for k in range(_NCHUNK):
        colbase = j * _BLK + k * _CHUNK
        idx = rb + colbase.astype(jnp.uint32)
        bits = _threefry_bits(idx)
        fb = jax.lax.shift_right_logical(bits, np.uint32(9)) | _ONE_BITS
        u = jax.lax.bitcast_convert_type(fb, jnp.float32) - np.float32(1.0)
        t = jnp.maximum(_TINY, u + _TINY)
        g = -jnp.log(-jnp.log(t))
        z = g + x_ref[:, k * _CHUNK : (k + 1) * _CHUNK]
        valid = lane_i32 < (_C - colbase)
        z = jnp.where(valid, z, _NEG_INF)
        col = lane_i32 + colbase
        upd = z > vm
        vm = jnp.where(upd, z, vm)
        vi = jnp.where(upd, col, vi)
    vmax_ref[...] = vm
    vidx_ref[...] = vi

    @pl.when(j == _NBLK - 1)
    def _finish():
        m = jnp.max(vm, axis=1, keepdims=True)
        cand = jnp.where(vm == m, vi, _INT_MAX)
        o_ref[...] = jnp.min(cand, axis=1, keepdims=True)


@functools.partial(jax.jit, static_argnames=("interpret",))
def kernel(logits, interpret=False):
    out = pl.pallas_call(
        _kernel,
        grid=(_NBLK,),
        in_specs=[pl.BlockSpec((_R, _BLK), lambda j: (0, j))],
        out_specs=pl.BlockSpec((_R, 1), lambda j: (0, 0)),
        out_shape=jax.ShapeDtypeStruct((_R, 1), jnp.int32),
        scratch_shapes=[
            pltpu.VMEM((_R, _CHUNK), jnp.float32),
            pltpu.VMEM((_R, _CHUNK), jnp.int32),
        ],
        interpret=interpret,
    )(logits)
    return out.reshape(_R)


# speedup vs baseline: 1.0720x; 1.0213x over previous
import functools

import jax
import jax.numpy as jnp
import numpy as np
from jax.experimental import pallas as pl
from jax.experimental.pallas import tpu as pltpu

_R, _C = 64, 100000
_BLK = 2048
_CHUNK = 128
_NBLK = (_C + _BLK - 1) // _BLK
_NCHUNK = _BLK // _CHUNK

_KS0 = 0
_KS1 = 42
_KS2 = (0x1BD11BDA ^ _KS0 ^ _KS1) & 0xFFFFFFFF
_TINY = np.float32(np.finfo(np.float32).tiny)
_NEG_INF = np.float32(-np.inf)
_ONE_BITS = np.uint32(0x3F800000)
_INT_MAX = np.int32(np.iinfo(np.int32).max)


def _rotl(x, d):
    return jax.lax.shift_left(x, np.uint32(d)) | jax.lax.shift_right_logical(
        x, np.uint32(32 - d)
    )


def _round(x0, x1, r):
    x0 = x0 + x1
    x1 = _rotl(x1, r)
    return x0, x1 ^ x0


def _threefry_bits(x1):
    x0 = x1
    x1 = _rotl(x1, 13) ^ x1
    x0, x1 = _round(x0, x1, 15)
    x0, x1 = _round(x0, x1, 26)
    x0, x1 = _round(x0, x1, 6)
    x0 = x0 + np.uint32(_KS1)
    x1 = x1 + np.uint32(_KS2 + 1)
    for r in (17, 29, 16, 24):
        x0, x1 = _round(x0, x1, r)
    x0 = x0 + np.uint32(_KS2)
    x1 = x1 + np.uint32(_KS0 + 2)
    for r in (13, 15, 26, 6):
        x0, x1 = _round(x0, x1, r)
    x1 = x1 + np.uint32(_KS1 + 3)
    for r in (17, 29, 16, 24):
        x0, x1 = _round(x0, x1, r)
    x0 = x0 + np.uint32(_KS1)
    x1 = x1 + np.uint32(_KS2 + 4)
    for r in (13, 15, 26, 6):
        x0, x1 = _round(x0, x1, r)
    x0 = x0 + np.uint32(_KS2)
    x1 = x1 + np.uint32(_KS0 + 5)
    return x0 ^ x1


def _kernel(x_ref, o_ref, vmax_ref, vidx_ref):
    j = pl.program_id(0)

    @pl.when(j == 0)
    def _init():
        vmax_ref[...] = jnp.full((_R, _CHUNK), _NEG_INF, jnp.float32)
        vidx_ref[...] = jnp.zeros((_R, _CHUNK), jnp.int32)

    row_iota = jax.lax.broadcasted_iota(jnp.uint32, (_R, _CHUNK), 0)
    lane_u32 = jax.lax.broadcasted_iota(jnp.uint32, (_R, _CHUNK), 1)
    rb = row_iota * np.uint32(_C) + lane_u32 + np.uint32(_KS1)
    lane_i32 = jax.lax.broadcasted_iota(jnp.int32, (_R, _CHUNK), 1)

    vm = vmax_ref[...]
    vi = vidx_ref[...]
    for k in range(_NCHUNK):
        colbase = j * _BLK + k * _CHUNK
        bits = _threefry_bits(rb + colbase.astype(jnp.uint32))
        fb = jax.lax.shift_right_logical(bits, np.uint32(9)) | _ONE_BITS
        u = jax.lax.bitcast_convert_type(fb, jnp.float32) - np.float32(1.0)
        t = jnp.maximum(u, _TINY)
        g = -jnp.log(-jnp.log(t))
        z = g + x_ref[:, k * _CHUNK : (k + 1) * _CHUNK]
        if k >= _NCHUNK - 3:
            z = jnp.where(lane_i32 < (_C - colbase), z, _NEG_INF)
        upd = z > vm
        vm = jnp.where(upd, z, vm)
        vi = jnp.where(upd, j * _NCHUNK + k, vi)
    vmax_ref[...] = vm
    vidx_ref[...] = vi

    @pl.when(j == _NBLK - 1)
    def _finish():
        col = vi * _CHUNK + lane_i32
        m = jnp.max(vm, axis=1, keepdims=True)
        cand = jnp.where(vm == m, col, _INT_MAX)
        o_ref[...] = jnp.min(cand, axis=1, keepdims=True)


@functools.partial(jax.jit, static_argnames=("interpret",))
def kernel(logits, interpret=False):
    out = pl.pallas_call(
        _kernel,
        grid=(_NBLK,),
        in_specs=[pl.BlockSpec((_R, _BLK), lambda j: (0, j))],
        out_specs=pl.BlockSpec((_R, 1), lambda j: (0, 0)),
        out_shape=jax.ShapeDtypeStruct((_R, 1), jnp.int32),
        scratch_shapes=[
            pltpu.VMEM((_R, _CHUNK), jnp.float32),
            pltpu.VMEM((_R, _CHUNK), jnp.int32),
        ],
        interpret=interpret,
    )(logits)
    return out.reshape(_R)
